# Initial kernel scaffold; baseline (speedup 1.0000x reference)
import jax
import jax.numpy as jnp
from jax.experimental import pallas as pl


def kernel(x, edge_index, batch, c0_w1, c0_b1, c0_g, c0_be, c0_w2, c0_b2, c1_w1, c1_b1, c1_g, c1_be, c1_w2, c1_b2, c2_w1, c2_b1, c2_g, c2_be, c2_w2, c2_b2, bn_g, bn_b, fc_w, fc_b):
    raise NotImplementedError("write your pallas kernel here")


# speedup vs baseline: 4.5683x; 4.5683x over previous
import functools

import jax
import jax.numpy as jnp
from jax import lax
from jax.experimental import pallas as pl
from jax.experimental.pallas import tpu as pltpu, tpu_sc as plsc

N = 10000
NP = 10240
D = 128
E = 320000
G = 64
L = 64

NC = 2
NS = 16
NW = NC * NS
EPT = E // NW
K = 80
NCHUNK = EPT // K
RPT = NP // NS

_mesh = plsc.VectorSubcoreMesh(core_axis_name="c", subcore_axis_name="s")


@functools.partial(
    pl.kernel,
    out_type=jax.ShapeDtypeStruct((NC, NP, D), jnp.float32),
    mesh=_mesh,
    scratch_types=[
        pltpu.VMEM((K,), jnp.int32),
        pltpu.VMEM((K,), jnp.int32),
        pltpu.VMEM((K, D), jnp.float32),
        pltpu.VMEM_SHARED((NP, D), jnp.float32),
        pltpu.SemaphoreType.DMA,
    ],
)
def _sc_aggregate(x_hbm, src_hbm, dst_hbm, out_hbm, src_v, dst_v, rows_v, agg_sh, sem):
    c = lax.axis_index("c")
    s = lax.axis_index("s")
    wid = s * NC + c
    base = wid * EPT

    r0 = s * RPT
    pltpu.sync_copy(x_hbm.at[pl.ds(r0, RPT)], agg_sh.at[pl.ds(r0, RPT)])
    plsc.subcore_barrier()

    def body(i, carry):
        off = pl.multiple_of(base + i * K, 8)
        pltpu.sync_copy(src_hbm.at[pl.ds(off, K)], src_v)
        pltpu.sync_copy(dst_hbm.at[pl.ds(off, K)], dst_v)
        pltpu.async_copy(x_hbm.at[src_v], rows_v, sem).wait()
        pltpu.sync_copy(rows_v, agg_sh.at[dst_v], add=True)
        return carry

    lax.fori_loop(0, NCHUNK, body, 0)
    plsc.subcore_barrier()

    pltpu.sync_copy(agg_sh.at[pl.ds(r0, RPT)], out_hbm.at[c].at[pl.ds(r0, RPT)])


R = 640
NBLK = NP // R
_BN_S = 1.0 / (1.0 + 1e-5) ** 0.5


def _mlp_body(p_ref, x_ref, w1_ref, b1_ref, g_ref, be_ref, w2_ref, b2_ref, out_ref):
    h = p_ref[0] + p_ref[1] - x_ref[...]
    u = jnp.dot(h, w1_ref[...], preferred_element_type=jnp.float32) + b1_ref[...]
    u = jnp.where(u >= 0, u, 0.2 * u)
    u = u * (g_ref[...] * _BN_S) + be_ref[...]
    v = jnp.dot(u, w2_ref[...], preferred_element_type=jnp.float32) + b2_ref[...]
    out_ref[...] = jnp.where(v >= 0, v, 0.2 * v)


_row_spec = pl.BlockSpec((R, D), lambda i: (i, 0))
_pair_spec = pl.BlockSpec((NC, R, D), lambda i: (0, i, 0))
_w_spec = pl.BlockSpec((D, D), lambda i: (0, 0))
_v_spec = pl.BlockSpec((1, D), lambda i: (0, 0))


def _mlp(p, x, w1, b1, g, be, w2, b2):
    return pl.pallas_call(
        _mlp_body,
        grid=(NBLK,),
        in_specs=[_pair_spec, _row_spec, _w_spec, _v_spec, _v_spec, _v_spec,
                  _w_spec, _v_spec],
        out_specs=_row_spec,
        out_shape=jax.ShapeDtypeStruct((NP, D), jnp.float32),
    )(p, x, w1, b1.reshape(1, D), g.reshape(1, D),
      be.reshape(1, D), w2, b2.reshape(1, D))


def _mlp_pool_body(p_ref, x_ref, batch_ref, w1_ref, b1_ref, g_ref, be_ref,
                   w2_ref, b2_ref, bng_ref, bnb_ref, fcw_ref, fcb_ref,
                   out_ref, acc_ref):
    i = pl.program_id(0)
    h = p_ref[0] + p_ref[1] - x_ref[...]
    u = jnp.dot(h, w1_ref[...], preferred_element_type=jnp.float32) + b1_ref[...]
    u = jnp.where(u >= 0, u, 0.2 * u)
    u = u * (g_ref[...] * _BN_S) + be_ref[...]
    v = jnp.dot(u, w2_ref[...], preferred_element_type=jnp.float32) + b2_ref[...]
    v = jnp.where(v >= 0, v, 0.2 * v)

    b = batch_ref[0, 0, :]
    oh = (b[:, None] == lax.broadcasted_iota(jnp.int32, (R, G), 1)).astype(jnp.float32)
    part = lax.dot_general(oh, v, (((0,), (0,)), ((), ())),
                           preferred_element_type=jnp.float32)

    @pl.when(i == 0)
    def _():
        acc_ref[...] = jnp.zeros_like(acc_ref)

    acc_ref[...] += part

    @pl.when(i == NBLK - 1)
    def _():
        pooled = acc_ref[...] * (bng_ref[...] * _BN_S) + bnb_ref[...]
        out_ref[...] = (jnp.dot(pooled, fcw_ref[...],
                                preferred_element_type=jnp.float32)
                        + fcb_ref[...])


def _mlp_pool(p, x, batch3, w1, b1, g, be, w2, b2, bng, bnb, fcw, fcb):
    return pl.pallas_call(
        _mlp_pool_body,
        grid=(NBLK,),
        in_specs=[_pair_spec, _row_spec,
                  pl.BlockSpec((1, 1, R), lambda i: (i, 0, 0)),
                  _w_spec, _v_spec, _v_spec, _v_spec, _w_spec, _v_spec,
                  pl.BlockSpec((1, D), lambda i: (0, 0)),
                  pl.BlockSpec((1, D), lambda i: (0, 0)),
                  pl.BlockSpec((D, L), lambda i: (0, 0)),
                  pl.BlockSpec((1, L), lambda i: (0, 0))],
        out_specs=pl.BlockSpec((G, L), lambda i: (0, 0)),
        out_shape=jax.ShapeDtypeStruct((G, L), jnp.float32),
        scratch_shapes=[pltpu.VMEM((G, D), jnp.float32)],
    )(p, x, batch3, w1, b1.reshape(1, D), g.reshape(1, D),
      be.reshape(1, D), w2, b2.reshape(1, D),
      bng.reshape(1, D), bnb.reshape(1, D), fcw, fcb.reshape(1, L))


def kernel(x, edge_index, batch, c0_w1, c0_b1, c0_g, c0_be, c0_w2, c0_b2,
           c1_w1, c1_b1, c1_g, c1_be, c1_w2, c1_b2,
           c2_w1, c2_b1, c2_g, c2_be, c2_w2, c2_b2,
           bn_g, bn_b, fc_w, fc_b):
    src = edge_index[0]
    dst = edge_index[1]
    xp = jnp.pad(x, ((0, NP - N), (0, 0)))
    batch3 = jnp.pad(batch, (0, NP - N), constant_values=G).reshape(NBLK, 1, R)

    p = _sc_aggregate(xp, src, dst)
    h = _mlp(p, xp, c0_w1, c0_b1, c0_g, c0_be, c0_w2, c0_b2)
    p = _sc_aggregate(h, src, dst)
    h = _mlp(p, h, c1_w1, c1_b1, c1_g, c1_be, c1_w2, c1_b2)
    p = _sc_aggregate(h, src, dst)
    out = _mlp_pool(p, h, batch3, c2_w1, c2_b1, c2_g, c2_be, c2_w2, c2_b2,
                    bn_g, bn_b, fc_w, fc_b)
    return out
